# Initial kernel scaffold; baseline (speedup 1.0000x reference)
import jax
import jax.numpy as jnp
from jax.experimental import pallas as pl


def kernel(hidden_states, weight, bias):
    raise NotImplementedError("write your pallas kernel here")


# speedup vs baseline: 3.1375x; 3.1375x over previous
import functools

import jax
import jax.numpy as jnp
from jax.experimental import pallas as pl
from jax.experimental.pallas import tpu as pltpu

_TOP_K = 8


def _router_block(h_ref, wt_ref, b_ref, scores_ref, idx_ref):
    h = h_ref[...]
    wt = wt_ref[...]
    logits = jnp.dot(h, wt, preferred_element_type=jnp.float32)
    logits = logits + b_ref[...]

    bt, e = logits.shape
    lane = jax.lax.broadcasted_iota(jnp.int32, (bt, e), 1)
    neg = jnp.float32(-3.0e38)

    vals = logits
    top_v = []
    top_i = []
    for _ in range(_TOP_K):
        m = jnp.max(vals, axis=1, keepdims=True)
        i = jnp.min(jnp.where(vals == m, lane, e), axis=1, keepdims=True)
        top_v.append(m)
        top_i.append(i)
        vals = jnp.where(lane == i, neg, vals)

    tv = jnp.concatenate(top_v, axis=1)
    ti = jnp.concatenate(top_i, axis=1)

    vmax = tv[:, :1]
    ex = jnp.exp(tv - vmax)
    denom = jnp.sum(ex, axis=1, keepdims=True)

    thresh = tv[:, _TOP_K - 1:_TOP_K]
    scores_ref[...] = jnp.where(
        logits >= thresh, jnp.exp(logits - vmax) / denom, 0.0
    )
    idx_ref[...] = ti


@functools.partial(jax.jit, static_argnames=())
def kernel(hidden_states, weight, bias):
    t, h = hidden_states.shape
    e = weight.shape[0]
    bt = 256
    grid = (t // bt,)

    wt = weight.T
    b2 = bias.reshape(1, e)

    scores, idx = pl.pallas_call(
        _router_block,
        grid=grid,
        in_specs=[
            pl.BlockSpec((bt, h), lambda i: (i, 0)),
            pl.BlockSpec((h, e), lambda i: (0, 0)),
            pl.BlockSpec((1, e), lambda i: (0, 0)),
        ],
        out_specs=[
            pl.BlockSpec((bt, e), lambda i: (i, 0)),
            pl.BlockSpec((bt, _TOP_K), lambda i: (i, 0)),
        ],
        out_shape=[
            jax.ShapeDtypeStruct((t, e), jnp.float32),
            jax.ShapeDtypeStruct((t, _TOP_K), jnp.int32),
        ],
    )(hidden_states, wt, b2)
    return scores, idx


# speedup vs baseline: 3.8160x; 1.2163x over previous
import functools

import jax
import jax.numpy as jnp
from jax.experimental import pallas as pl

_TOP_K = 8


def _router_block(h_ref, wt_ref, b_ref, scores_ref, idx_ref):
    h = h_ref[...]
    wt = wt_ref[...]
    logits = jnp.dot(h, wt, preferred_element_type=jnp.float32)
    logits = logits + b_ref[...]

    bt, e = logits.shape
    neg = jnp.float32(-3.0e38)

    vals = logits
    top_v = []
    for _ in range(_TOP_K):
        m = jnp.max(vals, axis=1, keepdims=True)
        top_v.append(m)
        vals = jnp.where(vals == m, neg, vals)
    tv = jnp.concatenate(top_v, axis=1)

    vmax = tv[:, :1]
    v8 = tv[:, _TOP_K - 1:_TOP_K]
    ex = jnp.exp(tv - vmax)
    denom = jnp.sum(ex, axis=1, keepdims=True)
    scores_ref[...] = jnp.where(
        logits >= v8, jnp.exp(logits - vmax) / denom, 0.0
    )

    rank = jnp.zeros((bt, e), jnp.float32)
    for j in range(_TOP_K):
        rank = rank + (tv[:, j:j + 1] > logits).astype(jnp.float32)
    in_top = logits >= v8

    lane = jax.lax.broadcasted_iota(jnp.int32, (bt, e), 1).astype(jnp.float32)
    zero = jnp.zeros((), jnp.float32)
    def pos_weight(r):
        w = jnp.where(r == 0, 262144.0,
            jnp.where(r == 1, 4096.0,
            jnp.where(r == 2, 64.0,
            jnp.where(r == 3, 1.0, zero))))
        return w

    w_lo = jnp.where(in_top, pos_weight(rank) * lane, zero)
    w_hi = jnp.where(in_top, pos_weight(rank - 4.0) * lane, zero)

    wcat = jnp.concatenate([w_lo, w_hi], axis=1)
    sel_lo = (jax.lax.broadcasted_iota(jnp.int32, (2 * e, 2), 0) < e)
    sel = jnp.where(
        sel_lo == (jax.lax.broadcasted_iota(jnp.int32, (2 * e, 2), 1) == 0),
        1.0, 0.0,
    ).astype(jnp.float32)
    packed = jnp.dot(wcat, sel, preferred_element_type=jnp.float32)
    p_lo = packed[:, :1]
    p_hi = packed[:, 1:2]

    digits = []
    for p in (p_lo, p_hi):
        d0 = jnp.floor(p * (1.0 / 262144.0))
        r0 = p - d0 * 262144.0
        d1 = jnp.floor(r0 * (1.0 / 4096.0))
        r1 = r0 - d1 * 4096.0
        d2 = jnp.floor(r1 * (1.0 / 64.0))
        d3 = r1 - d2 * 64.0
        digits += [d0, d1, d2, d3]
    idx_ref[...] = jnp.concatenate(digits, axis=1).astype(jnp.int32)


@functools.partial(jax.jit, static_argnames=())
def kernel(hidden_states, weight, bias):
    t, h = hidden_states.shape
    e = weight.shape[0]
    bt = 256
    grid = (t // bt,)

    wt = weight.T
    b2 = bias.reshape(1, e)

    scores, idx = pl.pallas_call(
        _router_block,
        grid=grid,
        in_specs=[
            pl.BlockSpec((bt, h), lambda i: (i, 0)),
            pl.BlockSpec((h, e), lambda i: (0, 0)),
            pl.BlockSpec((1, e), lambda i: (0, 0)),
        ],
        out_specs=[
            pl.BlockSpec((bt, e), lambda i: (i, 0)),
            pl.BlockSpec((bt, _TOP_K), lambda i: (i, 0)),
        ],
        out_shape=[
            jax.ShapeDtypeStruct((t, e), jnp.float32),
            jax.ShapeDtypeStruct((t, _TOP_K), jnp.int32),
        ],
    )(hidden_states, wt, b2)
    return scores, idx


# speedup vs baseline: 5.4021x; 1.4156x over previous
import functools

import jax
import jax.numpy as jnp
from jax.experimental import pallas as pl

_TOP_K = 8
_BSUB = 256
_NSUB = 4


def _topk_block(logits):
    bt, e = logits.shape
    neg = jnp.float32(-3.0e38)

    vals = logits
    top_v = []
    for _ in range(_TOP_K):
        m = jnp.max(vals, axis=1, keepdims=True)
        top_v.append(m)
        vals = jnp.where(vals == m, neg, vals)
    tv = jnp.concatenate(top_v, axis=1)

    vmax = tv[:, :1]
    v8 = tv[:, _TOP_K - 1:_TOP_K]
    ex = jnp.exp(tv - vmax)
    denom = jnp.sum(ex, axis=1, keepdims=True)
    scores = jnp.where(logits >= v8, jnp.exp(logits - vmax) / denom, 0.0)

    lane = jax.lax.broadcasted_iota(jnp.int32, (bt, e), 1).astype(jnp.float32)
    coef = (262144.0, 4096.0, 64.0, 1.0)
    s_lo = jnp.zeros((bt, e), jnp.float32)
    s_hi = jnp.zeros((bt, e), jnp.float32)
    for j in range(4):
        s_lo = s_lo + (logits == tv[:, j:j + 1]).astype(jnp.float32) * coef[j]
        s_hi = s_hi + (logits == tv[:, j + 4:j + 5]).astype(jnp.float32) * coef[j]
    w_lo = s_lo * lane
    w_hi = s_hi * lane

    wcat = jnp.concatenate([w_lo, w_hi], axis=1)
    sel_lo = (jax.lax.broadcasted_iota(jnp.int32, (2 * e, 2), 0) < e)
    sel = jnp.where(
        sel_lo == (jax.lax.broadcasted_iota(jnp.int32, (2 * e, 2), 1) == 0),
        1.0, 0.0,
    ).astype(jnp.float32)
    packed = jnp.dot(wcat, sel, preferred_element_type=jnp.float32)
    p_lo = packed[:, :1]
    p_hi = packed[:, 1:2]

    digits = []
    for p in (p_lo, p_hi):
        d0 = jnp.floor(p * (1.0 / 262144.0))
        r0 = p - d0 * 262144.0
        d1 = jnp.floor(r0 * (1.0 / 4096.0))
        r1 = r0 - d1 * 4096.0
        d2 = jnp.floor(r1 * (1.0 / 64.0))
        d3 = r1 - d2 * 64.0
        digits += [d0, d1, d2, d3]
    idx = jnp.concatenate(digits, axis=1).astype(jnp.int32)
    return scores, idx


def _router_block(h_ref, wt_ref, b_ref, scores_ref, idx_ref):
    wt = wt_ref[...]
    b2 = b_ref[...]
    logits_list = []
    for s in range(_NSUB):
        h = h_ref[pl.ds(s * _BSUB, _BSUB), :]
        lg = jnp.dot(h, wt, preferred_element_type=jnp.float32) + b2
        logits_list.append(lg)
    for s in range(_NSUB):
        scores, idx = _topk_block(logits_list[s])
        scores_ref[pl.ds(s * _BSUB, _BSUB), :] = scores
        idx_ref[pl.ds(s * _BSUB, _BSUB), :] = idx


@functools.partial(jax.jit, static_argnames=())
def kernel(hidden_states, weight, bias):
    t, h = hidden_states.shape
    e = weight.shape[0]
    bt = _BSUB * _NSUB
    grid = (t // bt,)

    wt = weight.T
    b2 = bias.reshape(1, e)

    scores, idx = pl.pallas_call(
        _router_block,
        grid=grid,
        in_specs=[
            pl.BlockSpec((bt, h), lambda i: (i, 0)),
            pl.BlockSpec((h, e), lambda i: (0, 0)),
            pl.BlockSpec((1, e), lambda i: (0, 0)),
        ],
        out_specs=[
            pl.BlockSpec((bt, e), lambda i: (i, 0)),
            pl.BlockSpec((bt, _TOP_K), lambda i: (i, 0)),
        ],
        out_shape=[
            jax.ShapeDtypeStruct((t, e), jnp.float32),
            jax.ShapeDtypeStruct((t, _TOP_K), jnp.int32),
        ],
    )(hidden_states, wt, b2)
    return scores, idx


# speedup vs baseline: 6.1131x; 1.1316x over previous
import functools

import jax
import jax.numpy as jnp
from jax.experimental import pallas as pl

_TOP_K = 8
_BSUB = 256
_NSUB = 4


def _topk_block(logits):
    bt, e = logits.shape
    neg = jnp.float32(-3.0e38)
    zero = jnp.zeros((), jnp.float32)
    coef = (262144.0, 4096.0, 64.0, 1.0)

    vals = logits
    s_lo = jnp.zeros((bt, e), jnp.float32)
    s_hi = jnp.zeros((bt, e), jnp.float32)
    vmax = None
    den = None
    v8 = None
    for j in range(_TOP_K):
        m = jnp.max(vals, axis=1, keepdims=True)
        eq = vals == m
        if j == 0:
            vmax = m
            den = jnp.ones((bt, 1), jnp.float32)
        else:
            den = den + jnp.exp(m - vmax)
        if j == _TOP_K - 1:
            v8 = m
        if j < 4:
            s_lo = s_lo + jnp.where(eq, coef[j], zero)
        else:
            s_hi = s_hi + jnp.where(eq, coef[j - 4], zero)
        if j < _TOP_K - 1:
            vals = jnp.where(eq, neg, vals)

    rden = 1.0 / den
    scores = jnp.where(logits >= v8, jnp.exp(logits - vmax) * rden, 0.0)

    lane = jax.lax.broadcasted_iota(jnp.int32, (bt, e), 1).astype(jnp.float32)
    w_lo = s_lo * lane
    w_hi = s_hi * lane

    wcat = jnp.concatenate([w_lo, w_hi], axis=1)
    sel_lo = (jax.lax.broadcasted_iota(jnp.int32, (2 * e, 2), 0) < e)
    sel = jnp.where(
        sel_lo == (jax.lax.broadcasted_iota(jnp.int32, (2 * e, 2), 1) == 0),
        1.0, 0.0,
    ).astype(jnp.float32)
    packed = jnp.dot(wcat, sel, preferred_element_type=jnp.float32)
    p_lo = packed[:, :1]
    p_hi = packed[:, 1:2]

    digits = []
    for p in (p_lo, p_hi):
        d0 = jnp.floor(p * (1.0 / 262144.0))
        r0 = p - d0 * 262144.0
        d1 = jnp.floor(r0 * (1.0 / 4096.0))
        r1 = r0 - d1 * 4096.0
        d2 = jnp.floor(r1 * (1.0 / 64.0))
        d3 = r1 - d2 * 64.0
        digits += [d0, d1, d2, d3]
    idx = jnp.concatenate(digits, axis=1).astype(jnp.int32)
    return scores, idx


def _router_block(h_ref, wt_ref, b_ref, scores_ref, idx_ref):
    wt = wt_ref[...]
    b2 = b_ref[...]

    def dot_sub(s):
        h = h_ref[pl.ds(s * _BSUB, _BSUB), :]
        return jnp.dot(h, wt, preferred_element_type=jnp.float32) + b2

    def emit_sub(s, lg):
        scores, idx = _topk_block(lg)
        scores_ref[pl.ds(s * _BSUB, _BSUB), :] = scores
        idx_ref[pl.ds(s * _BSUB, _BSUB), :] = idx

    lg = dot_sub(0)
    for s in range(_NSUB):
        lg_next = dot_sub(s + 1) if s + 1 < _NSUB else None
        emit_sub(s, lg)
        lg = lg_next


@functools.partial(jax.jit, static_argnames=())
def kernel(hidden_states, weight, bias):
    t, h = hidden_states.shape
    e = weight.shape[0]
    bt = _BSUB * _NSUB
    grid = (t // bt,)

    wt = weight.T
    b2 = bias.reshape(1, e)

    scores, idx = pl.pallas_call(
        _router_block,
        grid=grid,
        in_specs=[
            pl.BlockSpec((bt, h), lambda i: (i, 0)),
            pl.BlockSpec((h, e), lambda i: (0, 0)),
            pl.BlockSpec((1, e), lambda i: (0, 0)),
        ],
        out_specs=[
            pl.BlockSpec((bt, e), lambda i: (i, 0)),
            pl.BlockSpec((bt, _TOP_K), lambda i: (i, 0)),
        ],
        out_shape=[
            jax.ShapeDtypeStruct((t, e), jnp.float32),
            jax.ShapeDtypeStruct((t, _TOP_K), jnp.int32),
        ],
    )(hidden_states, wt, b2)
    return scores, idx
